# baseline (speedup 1.0000x reference)
import jax
import jax.numpy as jnp
from jax.experimental import pallas as pl
from jax.experimental.pallas import tpu as pltpu


def _router_matmul_kernel(xa_ref, xb_ref, w_ref, o_ref):
    half = xa_ref.shape[0]
    w = w_ref[...]
    o_ref[:half, :] = jnp.dot(xa_ref[...], w,
                              preferred_element_type=jnp.float32)
    o_ref[half:, :] = jnp.dot(xb_ref[...], w,
                              preferred_element_type=jnp.float32)


def kernel(hidden_states, W):
    M, K = hidden_states.shape
    K2, N = W.shape
    assert K == K2
    BM = 512
    grid = (M // (2 * BM),)
    return pl.pallas_call(
        _router_matmul_kernel,
        grid=grid,
        in_specs=[
            pl.BlockSpec((BM, K), lambda i: (2 * i, 0)),
            pl.BlockSpec((BM, K), lambda i: (2 * i + 1, 0)),
            pl.BlockSpec((K, N), lambda i: (0, 0)),
        ],
        out_specs=pl.BlockSpec((2 * BM, N), lambda i: (i, 0)),
        out_shape=jax.ShapeDtypeStruct((M, N), jnp.float32),
        compiler_params=pltpu.CompilerParams(
            dimension_semantics=("arbitrary",),
        ),
    )(hidden_states, hidden_states, W)
